# baseline (speedup 1.0000x reference)
import functools

import jax
import jax.numpy as jnp
from jax import lax
from jax.experimental import pallas as pl
from jax.experimental.pallas import tpu as pltpu
from jax.experimental.pallas import tpu_sc as plsc

_B, _V, _L = 1024, 100000, 50
_WV = 2048
_NJ = -(-_V // _WV)

_NC, _NS = 2, 16
_NW = _NC * _NS
_GW = _L + 1
_TPW = _B * _GW // _NW
_CH = 96
_NCH = _TPW // _CH


def _stats_body(predt_ref, m_ref, s_ref):
    j = pl.program_id(0)
    x = predt_ref[...]
    row = lax.broadcasted_iota(jnp.int32, x.shape, 0) + j * _WV
    x = jnp.where(row < _V, x, -jnp.inf)
    bmax = jnp.max(x, axis=0, keepdims=True)

    @pl.when(j == 0)
    def _():
        m_ref[...] = bmax
        s_ref[...] = jnp.sum(jnp.exp(x - bmax), axis=0, keepdims=True)

    @pl.when(j > 0)
    def _():
        m_old = m_ref[...]
        m_new = jnp.maximum(m_old, bmax)
        s_ref[...] = s_ref[...] * jnp.exp(m_old - m_new) + jnp.sum(
            jnp.exp(x - m_new), axis=0, keepdims=True)
        m_ref[...] = m_new


def _gather_body(predt_hbm, ridx_hbm, lidx_hbm, out_hbm,
                 ridx_v, lidx_v, buf_v, out_v, sem):
    wid = lax.axis_index("s") * _NC + lax.axis_index("c")
    pltpu.sync_copy(ridx_hbm.at[wid], ridx_v)
    pltpu.sync_copy(lidx_hbm.at[wid], lidx_v)
    iota16 = lax.iota(jnp.int32, 16)
    for c in range(_NCH):
        pltpu.async_copy(
            predt_hbm.at[ridx_v.at[pl.ds(c * _CH, _CH)]], buf_v, sem).wait()

        def grp_body(gi, _, c=c):
            t0 = c * _CH + gi * 16
            lvec = lidx_v[pl.ds(t0, 16)]
            res = jnp.zeros((16,), jnp.float32)
            for k in range(16):
                l_k = lvec[k]
                x16 = buf_v[gi * 16 + k, pl.ds((l_k // 16) * 16, 16)]
                e_vec = x16.at[jnp.full((16,), l_k % 16, jnp.int32)].get(
                    mode="promise_in_bounds")
                res = res + jnp.where(iota16 == k, e_vec, 0.0)
            out_v[pl.ds(t0, 16)] = res
            return 0

        lax.fori_loop(0, _CH // 16, grp_body, 0)
    pltpu.sync_copy(out_v, out_hbm.at[wid])


def _combine_body(valst_ref, seqt_ref, post_ref, m_ref, s_ref, out_ref):
    g = valst_ref[0:_L, :]
    p_pos = valst_ref[_L:_L + 1, :]
    seq = seqt_ref[...]
    pos = post_ref[...]
    m_full = m_ref[...]
    s_full = s_ref[...]

    rowid = lax.broadcasted_iota(jnp.int32, (_L, _B), 0)
    dup = jnp.zeros((_L, _B), jnp.bool_)
    for lp in range(_L - 1):
        dup = jnp.logical_or(
            dup, jnp.logical_and(seq == seq[lp:lp + 1, :], rowid > lp))
    uniq = jnp.logical_not(dup)

    in_hist = jnp.any(seq == pos, axis=0, keepdims=True)

    m_h = jnp.max(g, axis=0, keepdims=True)
    s_h = jnp.sum(jnp.where(uniq, jnp.exp(g - m_h), 0.0), axis=0,
                  keepdims=True)
    prob_rep = jnp.exp(p_pos - m_h) / s_h

    s_hist = jnp.sum(jnp.where(uniq, jnp.exp(g - m_full), 0.0), axis=0,
                     keepdims=True)
    denom = jnp.maximum(s_full - s_hist, jnp.float32(1e-30))
    prob_expl = jnp.exp(p_pos - m_full) / denom

    prob = jnp.where(in_hist, prob_rep, prob_expl)
    logp = jnp.log(prob + jnp.float32(1e-8))
    out_ref[...] = -jnp.sum(logp, keepdims=True) / _B


def _row_stats(pred_t):
    return pl.pallas_call(
        _stats_body,
        grid=(_NJ,),
        in_specs=[pl.BlockSpec((_WV, _B), lambda j: (j, 0))],
        out_specs=[
            pl.BlockSpec((1, _B), lambda j: (0, 0)),
            pl.BlockSpec((1, _B), lambda j: (0, 0)),
        ],
        out_shape=[
            jax.ShapeDtypeStruct((1, _B), jnp.float32),
            jax.ShapeDtypeStruct((1, _B), jnp.float32),
        ],
    )(pred_t)


@functools.cache
def _gather_call():
    return pl.kernel(
        _gather_body,
        mesh=plsc.VectorSubcoreMesh(core_axis_name="c", subcore_axis_name="s"),
        out_type=jax.ShapeDtypeStruct((_NW, _TPW), jnp.float32),
        scratch_types=[
            pltpu.VMEM((_TPW,), jnp.int32),
            pltpu.VMEM((_TPW,), jnp.int32),
            pltpu.VMEM((_CH, _B), jnp.float32),
            pltpu.VMEM((_TPW,), jnp.float32),
            pltpu.SemaphoreType.DMA,
        ],
    )


def _combine(vals, seq, pos, m, s):
    out = pl.pallas_call(
        _combine_body,
        out_shape=jax.ShapeDtypeStruct((1, 1), jnp.float32),
    )(vals, seq, pos, m, s)
    return out[0, 0]


def kernel(pred, pos_items, item_seq):
    v_t = jnp.concatenate(
        [item_seq, pos_items[:, None]], axis=1).T
    b_t = jnp.arange(_B, dtype=jnp.int32)[None, :]
    ridx = v_t.reshape(_NW, _TPW)
    lidx = jnp.broadcast_to(b_t, (_GW, _B)).reshape(_NW, _TPW)

    m, s = _row_stats(pred.T)
    vals_t = _gather_call()(pred.T, ridx, lidx).reshape(_GW, _B)

    return _combine(vals_t, item_seq.T, pos_items[None, :], m, s)


# speedup vs baseline: 1.0037x; 1.0037x over previous
import functools

import jax
import jax.numpy as jnp
from jax import lax
from jax.experimental import pallas as pl
from jax.experimental.pallas import tpu as pltpu
from jax.experimental.pallas import tpu_sc as plsc

_B, _V, _L = 1024, 100000, 50
_WV = 2048
_NJ = -(-_V // _WV)

_NC, _NS = 2, 16
_NW = _NC * _NS
_GW = _L + 1
_TPW = _B * _GW // _NW
_CH = 96
_NCH = _TPW // _CH

_LOG2E = 1.4426950408889634


def _e(x):
    return jnp.exp2(x * _LOG2E)


def _stats_body(predt_ref, m_ref, s_ref):
    j = pl.program_id(0)
    x = predt_ref[...]
    row = lax.broadcasted_iota(jnp.int32, x.shape, 0) + j * _WV
    x = jnp.where(row < _V, x, -jnp.inf)
    bmax = jnp.max(x, axis=0, keepdims=True)

    @pl.when(j == 0)
    def _():
        m_ref[...] = bmax
        s_ref[...] = jnp.sum(_e(x - bmax), axis=0, keepdims=True)

    @pl.when(j > 0)
    def _():
        m_old = m_ref[...]
        m_new = jnp.maximum(m_old, bmax)
        s_ref[...] = s_ref[...] * _e(m_old - m_new) + jnp.sum(
            _e(x - m_new), axis=0, keepdims=True)
        m_ref[...] = m_new


def _gather_body(predt_hbm, ridx_hbm, lidx_hbm, out_hbm,
                 ridx_v, lidx_v, buf_v, out_v, sem):
    wid = lax.axis_index("s") * _NC + lax.axis_index("c")
    pltpu.sync_copy(ridx_hbm.at[wid], ridx_v)
    pltpu.sync_copy(lidx_hbm.at[wid], lidx_v)
    iota16 = lax.iota(jnp.int32, 16)
    for c in range(_NCH):
        pltpu.async_copy(
            predt_hbm.at[ridx_v.at[pl.ds(c * _CH, _CH)]], buf_v, sem).wait()

        def grp_body(gi, _, c=c):
            t0 = c * _CH + gi * 16
            lvec = lidx_v[pl.ds(t0, 16)]
            res = jnp.zeros((16,), jnp.float32)
            for k in range(16):
                l_k = lvec[k]
                x16 = buf_v[gi * 16 + k, pl.ds((l_k // 16) * 16, 16)]
                e_vec = x16.at[jnp.full((16,), l_k % 16, jnp.int32)].get(
                    mode="promise_in_bounds")
                res = res + jnp.where(iota16 == k, e_vec, 0.0)
            out_v[pl.ds(t0, 16)] = res
            return 0

        lax.fori_loop(0, _CH // 16, grp_body, 0)
    pltpu.sync_copy(out_v, out_hbm.at[wid])


def _combine_body(valst_ref, seqt_ref, post_ref, m_ref, s_ref, out_ref):
    g = valst_ref[0:_L, :]
    p_pos = valst_ref[_L:_L + 1, :]
    seq = seqt_ref[...]
    pos = post_ref[...]
    m_full = m_ref[...]
    s_full = s_ref[...]

    rowid = lax.broadcasted_iota(jnp.int32, (_L, _B), 0)
    dup = jnp.zeros((_L, _B), jnp.bool_)
    for lp in range(_L - 1):
        dup = jnp.logical_or(
            dup, jnp.logical_and(seq == seq[lp:lp + 1, :], rowid > lp))
    uniq = jnp.logical_not(dup)

    in_hist = jnp.any(seq == pos, axis=0, keepdims=True)

    m_h = jnp.max(g, axis=0, keepdims=True)
    s_h = jnp.sum(jnp.where(uniq, _e(g - m_h), 0.0), axis=0,
                  keepdims=True)
    prob_rep = _e(p_pos - m_h) / s_h

    s_hist = jnp.sum(jnp.where(uniq, _e(g - m_full), 0.0), axis=0,
                     keepdims=True)
    denom = jnp.maximum(s_full - s_hist, jnp.float32(1e-30))
    prob_expl = _e(p_pos - m_full) / denom

    prob = jnp.where(in_hist, prob_rep, prob_expl)
    logp = jnp.log(prob + jnp.float32(1e-8))
    out_ref[...] = -jnp.sum(logp, keepdims=True) / _B


def _row_stats(pred_t):
    return pl.pallas_call(
        _stats_body,
        grid=(_NJ,),
        in_specs=[pl.BlockSpec((_WV, _B), lambda j: (j, 0))],
        out_specs=[
            pl.BlockSpec((1, _B), lambda j: (0, 0)),
            pl.BlockSpec((1, _B), lambda j: (0, 0)),
        ],
        out_shape=[
            jax.ShapeDtypeStruct((1, _B), jnp.float32),
            jax.ShapeDtypeStruct((1, _B), jnp.float32),
        ],
    )(pred_t)


@functools.cache
def _gather_call():
    return pl.kernel(
        _gather_body,
        mesh=plsc.VectorSubcoreMesh(core_axis_name="c", subcore_axis_name="s"),
        out_type=jax.ShapeDtypeStruct((_NW, _TPW), jnp.float32),
        scratch_types=[
            pltpu.VMEM((_TPW,), jnp.int32),
            pltpu.VMEM((_TPW,), jnp.int32),
            pltpu.VMEM((_CH, _B), jnp.float32),
            pltpu.VMEM((_TPW,), jnp.float32),
            pltpu.SemaphoreType.DMA,
        ],
    )


def _combine(vals, seq, pos, m, s):
    out = pl.pallas_call(
        _combine_body,
        out_shape=jax.ShapeDtypeStruct((1, 1), jnp.float32),
    )(vals, seq, pos, m, s)
    return out[0, 0]


def kernel(pred, pos_items, item_seq):
    v_t = jnp.concatenate(
        [item_seq, pos_items[:, None]], axis=1).T
    b_t = jnp.arange(_B, dtype=jnp.int32)[None, :]
    ridx = v_t.reshape(_NW, _TPW)
    lidx = jnp.broadcast_to(b_t, (_GW, _B)).reshape(_NW, _TPW)

    m, s = _row_stats(pred.T)
    vals_t = _gather_call()(pred.T, ridx, lidx).reshape(_GW, _B)

    return _combine(vals_t, item_seq.T, pos_items[None, :], m, s)
